# baseline (speedup 1.0000x reference)
import functools

import jax
import jax.numpy as jnp
from jax import lax
from jax.experimental import pallas as pl
from jax.experimental.pallas import tpu as pltpu
from jax.experimental.pallas import tpu_sc as plsc

NNZ = 41
BATCH = 64
NC = 2
NS = 16
NROWS = 4096
RPG = 8
GIDX = RPG * NNZ
LANES = 16
CB = BATCH // NC
CHUNKS = CB // LANES
RPS = NROWS // NS
GPS = RPS // RPG
WPS = RPS * NNZ
SUBDMA = ((0, 112), (112, 112), (224, 104))
WOFF = (0, 16, 25)


def _wchunk(j):
    if j < 32:
        return j // 16, j % 16
    return 2, j - 25


_mesh = plsc.VectorSubcoreMesh(core_axis_name="c", subcore_axis_name="s")


@functools.partial(
    pl.kernel,
    mesh=_mesh,
    out_type=jax.ShapeDtypeStruct((BATCH, NROWS), jnp.float32),
    compiler_params=pltpu.CompilerParams(use_tc_tiling_on_sc=False,
                                         needs_layout_passes=False),
    scratch_types=[
        pltpu.VMEM_SHARED((NROWS, CB), jnp.float32),
        pltpu.VMEM_SHARED((NROWS, CB), jnp.float32),
        pltpu.VMEM((WPS,), jnp.int32),
        pltpu.VMEM((WPS,), jnp.float32),
        pltpu.VMEM((WPS,), jnp.int32),
        pltpu.VMEM((WPS,), jnp.float32),
        pltpu.VMEM((GIDX, CB), jnp.float32),
        pltpu.VMEM((GIDX, CB), jnp.float32),
        pltpu.VMEM((RPS, CB), jnp.float32),
        pltpu.VMEM((CB, RPS), jnp.float32),
        pltpu.SemaphoreType.DMA,
        pltpu.SemaphoreType.DMA,
    ],
)
def _fused(x, bcols, bvals, acols, avals, out, x_s, t_s,
           bcols_v, bvals_v, acols_v, avals_v, buf0, buf1, tblk_v, yblk_v,
           sem0, sem1):
    cid = lax.axis_index("c")
    sid = lax.axis_index("s")
    c0 = cid * CB
    r0 = sid * RPS
    w0 = sid * WPS

    pltpu.sync_copy(x.at[pl.ds(r0, RPS), pl.ds(c0, CB)],
                    x_s.at[pl.ds(r0, RPS)])
    pltpu.sync_copy(bcols.at[pl.ds(w0, WPS)], bcols_v)
    pltpu.sync_copy(bvals.at[pl.ds(w0, WPS)], bvals_v)
    pltpu.sync_copy(acols.at[pl.ds(w0, WPS)], acols_v)
    pltpu.sync_copy(avals.at[pl.ds(w0, WPS)], avals_v)
    plsc.subcore_barrier()

    bufs = (buf0, buf1)
    sems = (sem0, sem1)

    def issue(tab_s, cols_v, g, buf, sem):
        for off, n in SUBDMA:
            pltpu.make_async_copy(
                tab_s.at[cols_v.at[pl.ds(g * GIDX + off, n)]],
                buf.at[pl.ds(off, n)], sem).start()

    def drain(tab_s, cols_v, g, buf, sem):
        for off, n in SUBDMA:
            pltpu.make_async_copy(
                tab_s.at[cols_v.at[pl.ds(g * GIDX + off, n)]],
                buf.at[pl.ds(off, n)], sem).wait()

    def run_stage(tab_s, cols_v, vals_v, store_row):
        issue(tab_s, cols_v, 0, buf0, sem0)

        def body(i, carry):
            for b in range(2):
                g = 2 * i + b
                buf, sem = bufs[b], sems[b]
                drain(tab_s, cols_v, g, buf, sem)

                nxt = g + 1

                @pl.when(nxt < GPS)
                def _():
                    issue(tab_s, cols_v, nxt, bufs[1 - b], sems[1 - b])

                for r in range(RPG):
                    acc = [jnp.zeros((LANES,), jnp.float32)
                           for _ in range(CHUNKS)]
                    vbase = g * GIDX + r * NNZ
                    vv = [vals_v[pl.ds(vbase + off, LANES)] for off in WOFF]
                    for j in range(NNZ):
                        ck, lane = _wchunk(j)
                        v = vv[ck][lane]
                        e = r * NNZ + j
                        for c in range(CHUNKS):
                            acc[c] = acc[c] + v * buf[e, pl.ds(c * LANES,
                                                               LANES)]
                    store_row(RPG * g + r, acc)
            return carry

        lax.fori_loop(0, GPS // 2, body, 0)

    def store_t(row, acc):
        for c in range(CHUNKS):
            tblk_v[row, pl.ds(c * LANES, LANES)] = acc[c]

    run_stage(x_s, bcols_v, bvals_v, store_t)
    pltpu.sync_copy(tblk_v, t_s.at[pl.ds(r0, RPS)])
    plsc.subcore_barrier()

    iota = lax.iota(jnp.int32, LANES)

    def store_y(row, acc):
        for c in range(CHUNKS):
            plsc.store_scatter(yblk_v, [c * LANES + iota,
                                        jnp.full((LANES,), row, jnp.int32)],
                               acc[c])

    run_stage(t_s, acols_v, avals_v, store_y)
    pltpu.sync_copy(yblk_v, out.at[pl.ds(c0, CB), pl.ds(r0, RPS)])


def kernel(x, a_row_ids, a_cols, a_vals, b_row_ids, b_cols, b_vals):
    y_t = _fused(x, b_cols, b_vals, a_cols, a_vals)
    return y_t[None, :, :]
